# Initial kernel scaffold; baseline (speedup 1.0000x reference)
import jax
import jax.numpy as jnp
from jax.experimental import pallas as pl


def kernel(x, adj, W0, b0, W1, b1, W2, b2):
    raise NotImplementedError("write your pallas kernel here")


# speedup vs baseline: 5.7764x; 5.7764x over previous
import functools

import jax
import jax.numpy as jnp
from jax.experimental import pallas as pl
from jax.experimental.pallas import tpu as pltpu

_BR_P = 200
_BR = 1000


def _prep_body(br, adj_ref, x_ref, w0_ref, abf_ref, dis_ref, z1_ref):
    i = pl.program_id(0)
    blk = adj_ref[...]
    rows = jax.lax.broadcasted_iota(jnp.int32, blk.shape, 0) + i * br
    cols = jax.lax.broadcasted_iota(jnp.int32, blk.shape, 1)
    blk = jnp.where(rows == cols, 1.0, blk)
    abf_ref[...] = blk.astype(jnp.bfloat16)
    dis = jax.lax.rsqrt(jnp.maximum(jnp.sum(blk, axis=1, keepdims=True), 1.0))
    dis_ref[...] = dis
    z = jnp.dot(x_ref[...], w0_ref[...], preferred_element_type=jnp.float32)
    z1_ref[...] = (dis * z).astype(jnp.bfloat16)


def _layer_body(emit_z, z_ref, dis_ref, b_ref, wn_ref, abf_ref, out_ref):
    acc = jnp.dot(abf_ref[...], z_ref[...], preferred_element_type=jnp.float32)
    h = jnp.maximum(acc * dis_ref[...] + b_ref[...], 0.0)
    if emit_z:
        z = jnp.dot(h, wn_ref[...], preferred_element_type=jnp.float32)
        out_ref[...] = (dis_ref[...] * z).astype(jnp.bfloat16)
    else:
        out_ref[...] = h


def _prep(adj, x, w0):
    n, f = x.shape
    return pl.pallas_call(
        functools.partial(_prep_body, _BR_P),
        grid=(n // _BR_P,),
        in_specs=[
            pl.BlockSpec((_BR_P, n), lambda i: (i, 0)),
            pl.BlockSpec((_BR_P, f), lambda i: (i, 0)),
            pl.BlockSpec((f, f), lambda i: (0, 0)),
        ],
        out_specs=[
            pl.BlockSpec((_BR_P, n), lambda i: (i, 0)),
            pl.BlockSpec((_BR_P, 1), lambda i: (i, 0)),
            pl.BlockSpec((_BR_P, f), lambda i: (i, 0)),
        ],
        out_shape=[
            jax.ShapeDtypeStruct((n, n), jnp.bfloat16),
            jax.ShapeDtypeStruct((n, 1), jnp.float32),
            jax.ShapeDtypeStruct((n, f), jnp.bfloat16),
        ],
        compiler_params=pltpu.CompilerParams(
            dimension_semantics=("arbitrary",)),
    )(adj, x, w0)


def _layer(z, dis, b2d, wn, abf, emit_z):
    n = abf.shape[0]
    f = z.shape[1]
    out_dtype = jnp.bfloat16 if emit_z else jnp.float32
    return pl.pallas_call(
        functools.partial(_layer_body, emit_z),
        grid=(n // _BR,),
        in_specs=[
            pl.BlockSpec((n, f), lambda i: (0, 0)),
            pl.BlockSpec((_BR, 1), lambda i: (i, 0)),
            pl.BlockSpec((1, f), lambda i: (0, 0)),
            pl.BlockSpec((f, f), lambda i: (0, 0)),
            pl.BlockSpec((_BR, n), lambda i: (i, 0)),
        ],
        out_specs=pl.BlockSpec((_BR, f), lambda i: (i, 0)),
        out_shape=jax.ShapeDtypeStruct((n, f), out_dtype),
        compiler_params=pltpu.CompilerParams(
            dimension_semantics=("arbitrary",)),
    )(z, dis, b2d, wn, abf)


def kernel(x, adj, W0, b0, W1, b1, W2, b2):
    abf, dis, z1 = _prep(adj, x, W0)
    z2 = _layer(z1, dis, b0.reshape(1, -1), W1, abf, emit_z=True)
    z3 = _layer(z2, dis, b1.reshape(1, -1), W2, abf, emit_z=True)
    h = _layer(z3, dis, b2.reshape(1, -1), W2, abf, emit_z=False)
    return h
